# Initial kernel scaffold; baseline (speedup 1.0000x reference)
import functools

import jax
import jax.numpy as jnp
from jax import lax
from jax.experimental import pallas as pl
from jax.experimental.pallas import tpu as pltpu
from jax.experimental.pallas import tpu_sc as plsc

N = 10000
E = 320000
D = 128

NC = 2
NS = 16
N_PAD = 10016
ROWS_PER_TILE = N_PAD // NS

CK = 128
CHUNKS = 2560
CH_PER_CORE = CHUNKS // NC
CH_PER_TILE = CH_PER_CORE // NS


def _linear(x, W, b):
    def body(x_ref, w_ref, b_ref, o_ref):
        o_ref[...] = lax.dot_general(
            x_ref[...], w_ref[...], (((1,), (1,)), ((), ())),
            preferred_element_type=jnp.float32) + b_ref[...]

    return pl.pallas_call(
        body,
        grid=(10,),
        in_specs=[
            pl.BlockSpec((1000, D), lambda i: (i, 0)),
            pl.BlockSpec((D, D), lambda i: (0, 0)),
            pl.BlockSpec((1, D), lambda i: (0, 0)),
        ],
        out_specs=pl.BlockSpec((1000, D), lambda i: (i, 0)),
        out_shape=jax.ShapeDtypeStruct((N, D), jnp.float32),
    )(x, W, b)


def _combine_relu_linear(p, cnt, W, b):
    def body(p_ref, c_ref, w_ref, b_ref, o_ref):
        s = p_ref[0] + p_ref[1]
        c = c_ref[0] + c_ref[1]
        m = s / jnp.maximum(c[:, 0:1], 1.0)
        h = jnp.maximum(m, 0.0)
        o_ref[...] = lax.dot_general(
            h, w_ref[...], (((1,), (1,)), ((), ())),
            preferred_element_type=jnp.float32) + b_ref[...]

    blk = N_PAD // 4
    return pl.pallas_call(
        body,
        grid=(4,),
        in_specs=[
            pl.BlockSpec((2, blk, D), lambda i: (0, i, 0)),
            pl.BlockSpec((2, blk, 16), lambda i: (0, i, 0)),
            pl.BlockSpec((D, D), lambda i: (0, 0)),
            pl.BlockSpec((1, D), lambda i: (0, 0)),
        ],
        out_specs=pl.BlockSpec((blk, D), lambda i: (i, 0)),
        out_shape=jax.ShapeDtypeStruct((N_PAD, D), jnp.float32),
    )(p, cnt, W, b)


def _combine_mean(p, cnt):
    def body(p_ref, c_ref, o_ref):
        s = p_ref[0] + p_ref[1]
        c = c_ref[0] + c_ref[1]
        o_ref[...] = s / jnp.maximum(c[:, 0:1], 1.0)

    blk = N_PAD // 4
    return pl.pallas_call(
        body,
        grid=(4,),
        in_specs=[
            pl.BlockSpec((2, blk, D), lambda i: (0, i, 0)),
            pl.BlockSpec((2, blk, 16), lambda i: (0, i, 0)),
        ],
        out_specs=pl.BlockSpec((blk, D), lambda i: (i, 0)),
        out_shape=jax.ShapeDtypeStruct((N_PAD, D), jnp.float32),
    )(p, cnt)


def _make_agg(with_count):
    mesh = plsc.VectorSubcoreMesh(core_axis_name="c", subcore_axis_name="s")

    out_type = [jax.ShapeDtypeStruct((NC, N_PAD, D), jnp.float32)]
    scratch = [
        pltpu.VMEM((CH_PER_TILE, CK), jnp.int32),
        pltpu.VMEM((CH_PER_TILE, CK), jnp.int32),
        pltpu.VMEM((CK, D), jnp.float32),
        pltpu.VMEM_SHARED((N_PAD, D), jnp.float32),
        pltpu.SemaphoreType.DMA,
    ]
    if with_count:
        out_type.append(jax.ShapeDtypeStruct((NC, N_PAD, 16), jnp.float32))
        scratch += [
            pltpu.VMEM((CK, 16), jnp.float32),
            pltpu.VMEM_SHARED((N_PAD, 16), jnp.float32),
        ]

    def body(*refs):
        if with_count:
            (h, srcc, dstc, z128, z16, ones,
             out_p, out_c,
             src_v, dst_v, rows_v, acc_sh, sem, ones_v, cnt_sh) = refs
        else:
            (h, srcc, dstc, z128,
             out_p,
             src_v, dst_v, rows_v, acc_sh, sem) = refs

        c = lax.axis_index("c")
        s = lax.axis_index("s")
        row_lo = s * ROWS_PER_TILE

        pltpu.sync_copy(z128, acc_sh.at[pl.ds(row_lo, ROWS_PER_TILE)])
        if with_count:
            pltpu.sync_copy(z16, cnt_sh.at[pl.ds(row_lo, ROWS_PER_TILE)])
            pltpu.sync_copy(ones, ones_v)

        base = c * CH_PER_CORE + s * CH_PER_TILE
        pltpu.sync_copy(srcc.at[pl.ds(base, CH_PER_TILE)], src_v)
        pltpu.sync_copy(dstc.at[pl.ds(base, CH_PER_TILE)], dst_v)

        plsc.subcore_barrier()

        def step(j, carry):
            pltpu.async_copy(h.at[src_v.at[j]], rows_v, sem).wait()
            pltpu.sync_copy(rows_v, acc_sh.at[dst_v.at[j]], add=True)
            if with_count:
                pltpu.sync_copy(ones_v, cnt_sh.at[dst_v.at[j]], add=True)
            return carry

        lax.fori_loop(0, CH_PER_TILE, step, 0)

        plsc.subcore_barrier()

        pltpu.sync_copy(acc_sh.at[pl.ds(row_lo, ROWS_PER_TILE)],
                        out_p.at[c, pl.ds(row_lo, ROWS_PER_TILE)])
        if with_count:
            pltpu.sync_copy(cnt_sh.at[pl.ds(row_lo, ROWS_PER_TILE)],
                            out_c.at[c, pl.ds(row_lo, ROWS_PER_TILE)])

    return functools.partial(
        pl.kernel,
        mesh=mesh,
        out_type=tuple(out_type) if with_count else out_type[0],
        scratch_types=scratch,
    )(body)


def kernel(x, edge_index, W1, b1, W2, b2):
    src = edge_index[0].astype(jnp.int32)
    dst = edge_index[1].astype(jnp.int32)
    pad = CHUNKS * CK - E
    src2 = jnp.concatenate([src, jnp.zeros((pad,), jnp.int32)]).reshape(CHUNKS, CK)
    dst2 = jnp.concatenate([dst, jnp.full((pad,), N, jnp.int32)]).reshape(CHUNKS, CK)

    z128 = jnp.zeros((ROWS_PER_TILE, D), jnp.float32)
    z16 = jnp.zeros((ROWS_PER_TILE, 16), jnp.float32)
    ones = jnp.ones((CK, 16), jnp.float32)

    b1r = b1.reshape(1, D)
    b2r = b2.reshape(1, D)

    h1 = _linear(x, W1, b1r)
    p1, cnt = _make_agg(True)(h1, src2, dst2, z128, z16, ones)
    h2 = _combine_relu_linear(p1, cnt, W2, b2r)
    p2 = _make_agg(False)(h2, src2, dst2, z128)
    out = _combine_mean(p2, cnt)
    return out[:N]


# speedup vs baseline: 3.0713x; 3.0713x over previous
import functools

import jax
import jax.numpy as jnp
from jax import lax
from jax.experimental import pallas as pl
from jax.experimental.pallas import tpu as pltpu
from jax.experimental.pallas import tpu_sc as plsc

N = 10000
E = 320000
D = 128

NC = 2
NS = 16
N_PAD = 10112
ROWS_PER_TILE = N_PAD // NS

CK = 128
CHUNKS = 2560
CH_PER_CORE = CHUNKS // NC
CH_PER_TILE = CH_PER_CORE // NS


def _linear(x, W, b):
    def body(x_ref, w_ref, b_ref, o_ref):
        o_ref[...] = lax.dot_general(
            x_ref[...], w_ref[...], (((1,), (1,)), ((), ())),
            preferred_element_type=jnp.float32) + b_ref[...]

    return pl.pallas_call(
        body,
        grid=(10,),
        in_specs=[
            pl.BlockSpec((1000, D), lambda i: (i, 0)),
            pl.BlockSpec((D, D), lambda i: (0, 0)),
            pl.BlockSpec((1, D), lambda i: (0, 0)),
        ],
        out_specs=pl.BlockSpec((1000, D), lambda i: (i, 0)),
        out_shape=jax.ShapeDtypeStruct((N, D), jnp.float32),
    )(x, W, b)


def _combine_relu_linear(p, cnt, W, b):
    def body(p_ref, c_ref, w_ref, b_ref, o_ref):
        s = p_ref[0] + p_ref[1]
        c = c_ref[0] + c_ref[1]
        m = s / jnp.maximum(c[:, 0:1], 1.0)
        h = jnp.maximum(m, 0.0)
        o_ref[...] = lax.dot_general(
            h, w_ref[...], (((1,), (1,)), ((), ())),
            preferred_element_type=jnp.float32) + b_ref[...]

    return pl.pallas_call(
        body,
        out_shape=jax.ShapeDtypeStruct((N_PAD, D), jnp.float32),
    )(p, cnt, W, b)


def _combine_mean(p, cnt):
    def body(p_ref, c_ref, o_ref):
        s = p_ref[0] + p_ref[1]
        c = c_ref[0] + c_ref[1]
        o_ref[...] = s / jnp.maximum(c[:, 0:1], 1.0)

    return pl.pallas_call(
        body,
        out_shape=jax.ShapeDtypeStruct((N_PAD, D), jnp.float32),
    )(p, cnt)


def _make_agg():
    mesh = plsc.VectorSubcoreMesh(core_axis_name="c", subcore_axis_name="s")

    scratch = [
        pltpu.VMEM((CH_PER_TILE, CK), jnp.int32),
        pltpu.VMEM((CH_PER_TILE, CK), jnp.int32),
        pltpu.VMEM((CK, D), jnp.float32),
        pltpu.VMEM_SHARED((N_PAD, D), jnp.float32),
        pltpu.SemaphoreType.DMA,
    ]

    def body(h, srcc, dstc, z128, out_p, src_v, dst_v, rows_v, acc_sh, sem):
        c = lax.axis_index("c")
        s = lax.axis_index("s")
        row_lo = s * ROWS_PER_TILE

        pltpu.sync_copy(z128, acc_sh.at[pl.ds(row_lo, ROWS_PER_TILE)])

        base = c * CH_PER_CORE + s * CH_PER_TILE
        pltpu.sync_copy(srcc.at[pl.ds(base, CH_PER_TILE)], src_v)
        pltpu.sync_copy(dstc.at[pl.ds(base, CH_PER_TILE)], dst_v)

        plsc.subcore_barrier()

        def step(j, carry):
            pltpu.async_copy(h.at[src_v.at[j]], rows_v, sem).wait()
            pltpu.sync_copy(rows_v, acc_sh.at[dst_v.at[j]], add=True)
            return carry

        lax.fori_loop(0, CH_PER_TILE, step, 0)

        plsc.subcore_barrier()

        pltpu.sync_copy(acc_sh.at[pl.ds(row_lo, ROWS_PER_TILE)],
                        out_p.at[c, pl.ds(row_lo, ROWS_PER_TILE)])

    return pl.kernel(
        body,
        mesh=mesh,
        out_type=jax.ShapeDtypeStruct((NC, N_PAD, D), jnp.float32),
        scratch_types=scratch,
    )


def _make_count():
    mesh = plsc.VectorSubcoreMesh(core_axis_name="c", subcore_axis_name="s")

    scratch = [
        pltpu.VMEM((CH_PER_TILE, CK), jnp.int32),
        pltpu.VMEM((CK, D), jnp.float32),
        pltpu.VMEM_SHARED((N_PAD, D), jnp.float32),
    ]

    def body(dstc, z128, ones, out_c, dst_v, ones_v, cnt_sh):
        c = lax.axis_index("c")
        s = lax.axis_index("s")
        row_lo = s * ROWS_PER_TILE

        pltpu.sync_copy(z128, cnt_sh.at[pl.ds(row_lo, ROWS_PER_TILE)])
        pltpu.sync_copy(ones, ones_v)
        base = c * CH_PER_CORE + s * CH_PER_TILE
        pltpu.sync_copy(dstc.at[pl.ds(base, CH_PER_TILE)], dst_v)

        plsc.subcore_barrier()

        def step(j, carry):
            pltpu.sync_copy(ones_v, cnt_sh.at[dst_v.at[j]], add=True)
            return carry

        lax.fori_loop(0, CH_PER_TILE, step, 0)

        plsc.subcore_barrier()

        pltpu.sync_copy(cnt_sh.at[pl.ds(row_lo, ROWS_PER_TILE)],
                        out_c.at[c, pl.ds(row_lo, ROWS_PER_TILE)])

    return pl.kernel(
        body,
        mesh=mesh,
        out_type=jax.ShapeDtypeStruct((NC, N_PAD, D), jnp.float32),
        scratch_types=scratch,
    )


def kernel(x, edge_index, W1, b1, W2, b2):
    src = edge_index[0].astype(jnp.int32)
    dst = edge_index[1].astype(jnp.int32)
    pad = CHUNKS * CK - E
    src2 = jnp.concatenate([src, jnp.zeros((pad,), jnp.int32)]).reshape(CHUNKS, CK)
    dst2 = jnp.concatenate([dst, jnp.full((pad,), N, jnp.int32)]).reshape(CHUNKS, CK)

    z128 = jnp.zeros((ROWS_PER_TILE, D), jnp.float32)
    ones = jnp.ones((CK, D), jnp.float32)

    b1r = b1.reshape(1, D)
    b2r = b2.reshape(1, D)

    cnt = _make_count()(dst2, z128, ones)
    h1 = _linear(x, W1, b1r)
    p1 = _make_agg()(h1, src2, dst2, z128)
    h2 = _combine_relu_linear(p1, cnt, W2, b2r)
    p2 = _make_agg()(h2, src2, dst2, z128)
    out = _combine_mean(p2, cnt)
    return out[:N]


# speedup vs baseline: 3.5501x; 1.1559x over previous
import functools

import jax
import jax.numpy as jnp
from jax import lax
from jax.experimental import pallas as pl
from jax.experimental.pallas import tpu as pltpu
from jax.experimental.pallas import tpu_sc as plsc

N = 10000
E = 320000
D = 128

NC = 2
NS = 16
N_PAD = 10112
ROWS_PER_TILE = N_PAD // NS

CK = 128
CHUNKS = 2560
CH_PER_CORE = CHUNKS // NC
CH_PER_TILE = CH_PER_CORE // NS
CH_PHASE = CH_PER_TILE // 2


def _linear(x, W, b):
    def body(x_ref, w_ref, b_ref, o_ref):
        o_ref[...] = lax.dot_general(
            x_ref[...], w_ref[...], (((1,), (1,)), ((), ())),
            preferred_element_type=jnp.float32) + b_ref[...]

    return pl.pallas_call(
        body,
        grid=(10,),
        in_specs=[
            pl.BlockSpec((1000, D), lambda i: (i, 0)),
            pl.BlockSpec((D, D), lambda i: (0, 0)),
            pl.BlockSpec((1, D), lambda i: (0, 0)),
        ],
        out_specs=pl.BlockSpec((1000, D), lambda i: (i, 0)),
        out_shape=jax.ShapeDtypeStruct((N, D), jnp.float32),
    )(x, W, b)


def _combine_relu_linear(p, cnt, W, b):
    def body(p_ref, c_ref, w_ref, b_ref, o_ref):
        s = p_ref[0] + p_ref[1]
        c = c_ref[0] + c_ref[1]
        m = s / jnp.maximum(c[:, 0:1], 1.0)
        h = jnp.maximum(m, 0.0)
        o_ref[...] = lax.dot_general(
            h, w_ref[...], (((1,), (1,)), ((), ())),
            preferred_element_type=jnp.float32) + b_ref[...]

    return pl.pallas_call(
        body,
        out_shape=jax.ShapeDtypeStruct((N_PAD, D), jnp.float32),
    )(p, cnt, W, b)


def _combine_mean(p, cnt):
    def body(p_ref, c_ref, o_ref):
        s = p_ref[0] + p_ref[1]
        c = c_ref[0] + c_ref[1]
        o_ref[...] = s / jnp.maximum(c[:, 0:1], 1.0)

    return pl.pallas_call(
        body,
        out_shape=jax.ShapeDtypeStruct((N_PAD, D), jnp.float32),
    )(p, cnt)


def _make_agg():
    mesh = plsc.VectorSubcoreMesh(core_axis_name="c", subcore_axis_name="s")

    NBUF = 2

    scratch = [
        pltpu.VMEM((CH_PHASE, CK), jnp.int32),
        pltpu.VMEM((CH_PHASE, CK), jnp.int32),
        pltpu.VMEM_SHARED((N_PAD, D), jnp.float32),
    ] + [pltpu.VMEM((CK, D), jnp.float32) for _ in range(NBUF)] \
      + [pltpu.SemaphoreType.DMA for _ in range(NBUF)]

    def body(h, srcc, dstc, z128, out_p, src_v, dst_v, acc_sh, *bufs):
        rows = bufs[:NBUF]
        sems = bufs[NBUF:]
        c = lax.axis_index("c")
        s = lax.axis_index("s")
        row_lo = s * ROWS_PER_TILE

        pltpu.sync_copy(z128, acc_sh.at[pl.ds(row_lo, ROWS_PER_TILE)])
        plsc.subcore_barrier()

        wid = c * NS + s
        for phase in range(2):
            pltpu.sync_copy(srcc.at[wid, pl.ds(phase * CH_PHASE, CH_PHASE)],
                            src_v)
            pltpu.sync_copy(dstc.at[wid, pl.ds(phase * CH_PHASE, CH_PHASE)],
                            dst_v)

            for b in range(NBUF):
                pltpu.async_copy(h.at[src_v.at[b]], rows[b], sems[b])

            def step(t, carry):
                for b in range(NBUF):
                    j = t * NBUF + b
                    pltpu.make_async_copy(h.at[src_v.at[j]], rows[b],
                                          sems[b]).wait()
                    pltpu.sync_copy(rows[b], acc_sh.at[dst_v.at[j]], add=True)
                    nj = j + NBUF

                    @pl.when(nj < CH_PHASE)
                    def _():
                        pltpu.async_copy(h.at[src_v.at[nj]], rows[b], sems[b])
                return carry

            lax.fori_loop(0, CH_PHASE // NBUF, step, 0)

        plsc.subcore_barrier()

        pltpu.sync_copy(acc_sh.at[pl.ds(row_lo, ROWS_PER_TILE)],
                        out_p.at[c, pl.ds(row_lo, ROWS_PER_TILE)])

    return pl.kernel(
        body,
        mesh=mesh,
        out_type=jax.ShapeDtypeStruct((NC, N_PAD, D), jnp.float32),
        scratch_types=scratch,
    )


def _make_count():
    mesh = plsc.VectorSubcoreMesh(core_axis_name="c", subcore_axis_name="s")

    scratch = [
        pltpu.VMEM((CH_PER_TILE, CK), jnp.int32),
        pltpu.VMEM((CK, D), jnp.float32),
        pltpu.VMEM_SHARED((N_PAD, D), jnp.float32),
    ]

    def body(dstc, z128, ones, out_c, dst_v, ones_v, cnt_sh):
        c = lax.axis_index("c")
        s = lax.axis_index("s")
        row_lo = s * ROWS_PER_TILE

        pltpu.sync_copy(z128, cnt_sh.at[pl.ds(row_lo, ROWS_PER_TILE)])
        pltpu.sync_copy(ones, ones_v)
        pltpu.sync_copy(dstc.at[c * NS + s], dst_v)

        plsc.subcore_barrier()

        def step(j, carry):
            pltpu.sync_copy(ones_v, cnt_sh.at[dst_v.at[j]], add=True)
            return carry

        lax.fori_loop(0, CH_PER_TILE, step, 0)

        plsc.subcore_barrier()

        pltpu.sync_copy(cnt_sh.at[pl.ds(row_lo, ROWS_PER_TILE)],
                        out_c.at[c, pl.ds(row_lo, ROWS_PER_TILE)])

    return pl.kernel(
        body,
        mesh=mesh,
        out_type=jax.ShapeDtypeStruct((NC, N_PAD, D), jnp.float32),
        scratch_types=scratch,
    )


def kernel(x, edge_index, W1, b1, W2, b2):
    src = edge_index[0].astype(jnp.int32)
    dst = edge_index[1].astype(jnp.int32)
    pad = CHUNKS * CK - E
    src2 = jnp.concatenate([src, jnp.zeros((pad,), jnp.int32)]).reshape(
        NC * NS, CH_PER_TILE, CK)
    dst2 = jnp.concatenate([dst, jnp.full((pad,), N, jnp.int32)]).reshape(
        NC * NS, CH_PER_TILE, CK)

    z128 = jnp.zeros((ROWS_PER_TILE, D), jnp.float32)
    ones = jnp.ones((CK, D), jnp.float32)

    b1r = b1.reshape(1, D)
    b2r = b2.reshape(1, D)

    cnt = _make_count()(dst2, z128, ones)
    h1 = _linear(x, W1, b1r)
    p1 = _make_agg()(h1, src2, dst2, z128)
    h2 = _combine_relu_linear(p1, cnt, W2, b2r)
    p2 = _make_agg()(h2, src2, dst2, z128)
    out = _combine_mean(p2, cnt)
    return out[:N]


# speedup vs baseline: 10.3842x; 2.9250x over previous
import functools

import jax
import jax.numpy as jnp
from jax import lax
from jax.experimental import pallas as pl
from jax.experimental.pallas import tpu as pltpu
from jax.experimental.pallas import tpu_sc as plsc

N = 10000
E = 320000
D = 128

NC = 2
NS = 16
N_PAD = 10112
ROWS_PER_TILE = N_PAD // NS

CK = 128
CHUNKS = 2560
CH_PER_CORE = CHUNKS // NC
CH_PER_TILE = CH_PER_CORE // NS
CH_PHASE = CH_PER_TILE // 2


def _linear(x, W, b):
    def body(x_ref, w_ref, b_ref, o_ref):
        o_ref[...] = lax.dot_general(
            x_ref[...], w_ref[...], (((1,), (1,)), ((), ())),
            preferred_element_type=jnp.float32) + b_ref[...]

    return pl.pallas_call(
        body,
        grid=(10,),
        in_specs=[
            pl.BlockSpec((1000, D), lambda i: (i, 0)),
            pl.BlockSpec((D, D), lambda i: (0, 0)),
            pl.BlockSpec((1, D), lambda i: (0, 0)),
        ],
        out_specs=pl.BlockSpec((1000, D), lambda i: (i, 0)),
        out_shape=jax.ShapeDtypeStruct((N, D), jnp.float32),
    )(x, W, b)


def _combine_relu_linear(p, cnt, W, b):
    def body(p_ref, c_ref, w_ref, b_ref, o_ref):
        s = p_ref[0] + p_ref[1]
        c = c_ref[0] + c_ref[1]
        m = s / jnp.maximum(c[:, 0:1], 1.0)
        h = jnp.maximum(m, 0.0)
        o_ref[...] = lax.dot_general(
            h, w_ref[...], (((1,), (1,)), ((), ())),
            preferred_element_type=jnp.float32) + b_ref[...]

    return pl.pallas_call(
        body,
        out_shape=jax.ShapeDtypeStruct((N_PAD, D), jnp.float32),
    )(p, cnt, W, b)


def _combine_mean(p, cnt):
    def body(p_ref, c_ref, o_ref):
        s = p_ref[0] + p_ref[1]
        c = c_ref[0] + c_ref[1]
        o_ref[...] = s / jnp.maximum(c[:, 0:1], 1.0)

    return pl.pallas_call(
        body,
        out_shape=jax.ShapeDtypeStruct((N_PAD, D), jnp.float32),
    )(p, cnt)


def _make_agg():
    mesh = plsc.VectorSubcoreMesh(core_axis_name="c", subcore_axis_name="s")

    NBUF = 2

    scratch = [
        pltpu.VMEM((CH_PHASE, CK), jnp.int32),
        pltpu.VMEM((CH_PHASE, CK), jnp.int32),
        pltpu.VMEM_SHARED((N_PAD, D), jnp.float32),
    ] + [pltpu.VMEM((CK, D), jnp.float32) for _ in range(NBUF)] \
      + [pltpu.SemaphoreType.DMA for _ in range(NBUF)]

    def body(h, srcc, dstc, z128, out_p, src_v, dst_v, acc_sh, *bufs):
        rows = bufs[:NBUF]
        sems = bufs[NBUF:]
        c = lax.axis_index("c")
        s = lax.axis_index("s")
        row_lo = s * ROWS_PER_TILE

        pltpu.sync_copy(z128, acc_sh.at[pl.ds(row_lo, ROWS_PER_TILE)])
        plsc.subcore_barrier()

        wid = c * NS + s
        for phase in range(2):
            pltpu.sync_copy(srcc.at[wid, pl.ds(phase * CH_PHASE, CH_PHASE)],
                            src_v)
            pltpu.sync_copy(dstc.at[wid, pl.ds(phase * CH_PHASE, CH_PHASE)],
                            dst_v)

            for b in range(NBUF):
                pltpu.async_copy(h.at[src_v.at[b]], rows[b], sems[b])

            def step(t, carry):
                for b in range(NBUF):
                    j = t * NBUF + b
                    pltpu.make_async_copy(h.at[src_v.at[j]], rows[b],
                                          sems[b]).wait()
                    pltpu.sync_copy(rows[b], acc_sh.at[dst_v.at[j]], add=True)
                    nj = j + NBUF

                    @pl.when(nj < CH_PHASE)
                    def _():
                        pltpu.async_copy(h.at[src_v.at[nj]], rows[b], sems[b])
                return carry

            lax.fori_loop(0, CH_PHASE // NBUF, step, 0)

        plsc.subcore_barrier()

        pltpu.sync_copy(acc_sh.at[pl.ds(row_lo, ROWS_PER_TILE)],
                        out_p.at[c, pl.ds(row_lo, ROWS_PER_TILE)])

    return pl.kernel(
        body,
        mesh=mesh,
        out_type=jax.ShapeDtypeStruct((NC, N_PAD, D), jnp.float32),
        scratch_types=scratch,
    )


def _make_count():
    mesh = plsc.VectorSubcoreMesh(core_axis_name="c", subcore_axis_name="s")

    scratch = [
        pltpu.VMEM((CH_PER_TILE, CK), jnp.int32),
        pltpu.VMEM((CK, D), jnp.float32),
        pltpu.VMEM_SHARED((N_PAD, D), jnp.float32),
    ]

    def body(dstc, z128, ones, out_c, dst_v, ones_v, cnt_sh):
        c = lax.axis_index("c")
        s = lax.axis_index("s")
        row_lo = s * ROWS_PER_TILE

        pltpu.sync_copy(z128, cnt_sh.at[pl.ds(row_lo, ROWS_PER_TILE)])
        pltpu.sync_copy(ones, ones_v)
        pltpu.sync_copy(dstc.at[c * NS + s], dst_v)

        plsc.subcore_barrier()

        def step(j, carry):
            pltpu.sync_copy(ones_v, cnt_sh.at[dst_v.at[j]], add=True)
            return carry

        lax.fori_loop(0, CH_PER_TILE, step, 0)

        plsc.subcore_barrier()

        pltpu.sync_copy(cnt_sh.at[pl.ds(row_lo, ROWS_PER_TILE)],
                        out_c.at[c, pl.ds(row_lo, ROWS_PER_TILE)])

    return pl.kernel(
        body,
        mesh=mesh,
        out_type=jax.ShapeDtypeStruct((NC, N_PAD, D), jnp.float32),
        scratch_types=scratch,
    )


def kernel(x, edge_index, W1, b1, W2, b2):
    src = edge_index[0].astype(jnp.int32)
    dst = edge_index[1].astype(jnp.int32)
    pad = CHUNKS * CK - E
    pad_i = jnp.arange(pad, dtype=jnp.int32)
    src2 = jnp.concatenate([src, pad_i % N]).reshape(
        NC * NS, CH_PER_TILE, CK)
    dst2 = jnp.concatenate([dst, N + pad_i % (N_PAD - N)]).reshape(
        NC * NS, CH_PER_TILE, CK)

    z128 = jnp.zeros((ROWS_PER_TILE, D), jnp.float32)
    ones = jnp.ones((CK, D), jnp.float32)

    b1r = b1.reshape(1, D)
    b2r = b2.reshape(1, D)

    cnt = _make_count()(dst2, z128, ones)
    h1 = _linear(x, W1, b1r)
    p1 = _make_agg()(h1, src2, dst2, z128)
    h2 = _combine_relu_linear(p1, cnt, W2, b2r)
    p2 = _make_agg()(h2, src2, dst2, z128)
    out = _combine_mean(p2, cnt)
    return out[:N]
